# baseline (speedup 1.0000x reference)
import functools

import jax
import jax.numpy as jnp
from jax import lax
from jax.experimental import pallas as pl
from jax.experimental.pallas import tpu as pltpu
from jax.experimental.pallas import tpu_sc as plsc

N_VN = 10000
N_CN = 5000
VN_DEG = 3
CN_DEG = 6
N_EDGES = N_VN * VN_DEG
BATCH = 1024
NUM_ITER = 5
NUM_BITS_PER_SYMBOL = 2
CODERATE = 1.0 - N_CN / N_VN

_PHI_LO = 8.5e-8
_PHI_HI = 16.635532

NW = 32
VPW = 320
CHUNK = 16
NCHUNK = VPW // CHUNK
N_PAD = NW * VPW
M_PAD = 5120
E_PAD = CN_DEG * M_PAD
M_BLK = 256

LANES = 16
BATCH_H = BATCH // 2
NQ = BATCH_H // LANES


def _phi(x):
    x = jnp.clip(x, _PHI_LO, _PHI_HI)
    return -jnp.log(jnp.tanh(x * 0.5))


def _cn_body(z_ref, w_ref, y_ref):
    m = z_ref[...] * w_ref[...]
    mag = _phi(jnp.abs(m))
    sgn = jnp.where(m < 0.0, -1.0, 1.0)
    mag_sum = jnp.sum(mag, axis=0, keepdims=True)
    sign_tot = sgn[0:1]
    for j in range(1, CN_DEG):
        sign_tot = sign_tot * sgn[j:j + 1]
    y_ref[...] = (sign_tot * sgn) * _phi(mag_sum - mag)


def _cn_update(z3, w3):
    grid = M_PAD // M_BLK
    return pl.pallas_call(
        _cn_body,
        grid=(grid,),
        in_specs=[
            pl.BlockSpec((CN_DEG, M_BLK, BATCH_H), lambda i: (0, i, 0)),
            pl.BlockSpec((CN_DEG, M_BLK, 1), lambda i: (0, i, 0)),
        ],
        out_specs=pl.BlockSpec((CN_DEG, M_BLK, BATCH_H), lambda i: (0, i, 0)),
        out_shape=jax.ShapeDtypeStruct((CN_DEG, M_PAD, BATCH_H), jnp.float32),
    )(z3, w3)


def _vn_body(y_hbm, l_hbm, idx_hbm, z_hbm, t_hbm,
             gin0, gin1, gout0, gout1, lt0, lt1, idx_v,
             sg0, sg1, sl0, sl1, ss0, ss1):
    wid = lax.axis_index("s") * 2 + lax.axis_index("c")
    pltpu.sync_copy(idx_hbm.at[wid], idx_v)
    gin = (gin0, gin1)
    gout = (gout0, gout1)
    lt = (lt0, lt1)
    sg = (sg0, sg1)
    sl = (sl0, sl1)
    ss = (ss0, ss1)
    base0 = wid * VPW

    def issue(k, b):
        pltpu.async_copy(y_hbm.at[idx_v.at[k]], gin[b], sg[b])
        pltpu.async_copy(l_hbm.at[pl.ds(base0 + k * CHUNK, CHUNK)], lt[b], sl[b])

    for b in range(2):
        issue(b, b)

    def super_body(s, carry):
        for b in range(2):
            k = 2 * s + b
            pltpu.make_async_copy(y_hbm.at[idx_v.at[k]], gin[b], sg[b]).wait()
            pltpu.make_async_copy(
                l_hbm.at[pl.ds(base0 + k * CHUNK, CHUNK)], lt[b], sl[b]).wait()

            @pl.when(s > 0)
            def _():
                pltpu.make_async_copy(gout[b], z_hbm.at[idx_v.at[k]], ss[b]).wait()

            def q_body(q, c2):
                col = pl.ds(q * LANES, LANES)
                for vl in range(CHUNK):
                    m0 = gin[b][3 * vl + 0, col]
                    m1 = gin[b][3 * vl + 1, col]
                    m2 = gin[b][3 * vl + 2, col]
                    t = lt[b][vl, col] + (m0 + m1 + m2)
                    lt[b][vl, col] = t
                    gout[b][3 * vl + 0, col] = t - m0
                    gout[b][3 * vl + 1, col] = t - m1
                    gout[b][3 * vl + 2, col] = t - m2
                return c2

            lax.fori_loop(0, NQ, q_body, 0)
            pltpu.sync_copy(lt[b], t_hbm.at[pl.ds(base0 + k * CHUNK, CHUNK)])
            pltpu.async_copy(gout[b], z_hbm.at[idx_v.at[k]], ss[b])

            @pl.when(k + 2 < NCHUNK)
            def _():
                issue(k + 2, b)
        return carry

    lax.fori_loop(0, NCHUNK // 2, super_body, 0)
    for b in range(2):
        k = NCHUNK - 2 + b
        pltpu.make_async_copy(gout[b], z_hbm.at[idx_v.at[k]], ss[b]).wait()


@functools.lru_cache(maxsize=None)
def _make_vn_update():
    return functools.partial(
        pl.kernel,
        out_type=(
            jax.ShapeDtypeStruct((E_PAD, BATCH_H), jnp.float32),
            jax.ShapeDtypeStruct((N_PAD, BATCH_H), jnp.float32),
        ),
        mesh=plsc.VectorSubcoreMesh(core_axis_name="c", subcore_axis_name="s"),
        scratch_types=[
            pltpu.VMEM((VN_DEG * CHUNK, BATCH_H), jnp.float32),
            pltpu.VMEM((VN_DEG * CHUNK, BATCH_H), jnp.float32),
            pltpu.VMEM((VN_DEG * CHUNK, BATCH_H), jnp.float32),
            pltpu.VMEM((VN_DEG * CHUNK, BATCH_H), jnp.float32),
            pltpu.VMEM((CHUNK, BATCH_H), jnp.float32),
            pltpu.VMEM((CHUNK, BATCH_H), jnp.float32),
            pltpu.VMEM((NCHUNK, VN_DEG * CHUNK), jnp.int32),
            pltpu.SemaphoreType.DMA,
            pltpu.SemaphoreType.DMA,
            pltpu.SemaphoreType.DMA,
            pltpu.SemaphoreType.DMA,
            pltpu.SemaphoreType.DMA,
            pltpu.SemaphoreType.DMA,
        ],
    )(_vn_body)


def kernel(w_re, w_im, edge_weights, ebno_db, edge_vn, edge_cn):
    no = 1.0 / (10.0 ** (ebno_db / 10.0) * NUM_BITS_PER_SYMBOL * CODERATE)

    e32 = jnp.arange(N_EDGES, dtype=jnp.int32)
    order = jnp.argsort(edge_cn.astype(jnp.int32), stable=True)
    rank = jnp.zeros((N_EDGES,), jnp.int32).at[order].set(e32)
    slot = (rank % CN_DEG) * M_PAD + rank // CN_DEG
    n_pad_e = VN_DEG * (N_PAD - N_VN)
    tpe = jnp.arange(n_pad_e, dtype=jnp.int32)
    pad_slot = (tpe // (M_PAD - N_CN)) * M_PAD + N_CN + tpe % (M_PAD - N_CN)
    gidx = jnp.concatenate([slot, pad_slot]).reshape(NW, NCHUNK, VN_DEG * CHUNK)

    w_flat = jnp.zeros((E_PAD,), jnp.float32).at[slot].set(edge_weights)
    w3 = w_flat.reshape(CN_DEG, M_PAD, 1)
    v_of_slot = jnp.zeros((E_PAD,), jnp.int32).at[slot].set(edge_vn.astype(jnp.int32))

    vn_update = _make_vn_update()
    l_halves, z_halves, t_halves = [], [], []
    for h in range(2):
        llr_h = (4.0 * (1.0 + w_re[h * BATCH_H:(h + 1) * BATCH_H]) / no).T
        l_h = jnp.pad(llr_h, ((0, N_PAD - N_VN), (0, 0)))
        l_halves.append(l_h)
        z_halves.append(jnp.take(l_h, v_of_slot, axis=0))
        t_halves.append(l_h)

    loss = jnp.float32(0.0)
    for _ in range(NUM_ITER):
        for h in range(2):
            y = _cn_update(z_halves[h].reshape(CN_DEG, M_PAD, BATCH_H), w3)
            z_halves[h], t_halves[h] = vn_update(
                y.reshape(E_PAD, BATCH_H), l_halves[h], gidx)
        loss = loss + 0.5 * (jnp.mean(jax.nn.softplus(-t_halves[0][:N_VN]))
                             + jnp.mean(jax.nn.softplus(-t_halves[1][:N_VN])))
    loss = loss / NUM_ITER

    c = jnp.zeros((BATCH, N_VN), dtype=jnp.float32)
    c_hat = jnp.concatenate([-t_halves[0][:N_VN].T, -t_halves[1][:N_VN].T], axis=0)
    return (c, c_hat, loss)


# speedup vs baseline: 1.3587x; 1.3587x over previous
import functools

import jax
import jax.numpy as jnp
from jax import lax
from jax.experimental import pallas as pl
from jax.experimental.pallas import tpu as pltpu
from jax.experimental.pallas import tpu_sc as plsc

N_VN = 10000
N_CN = 5000
VN_DEG = 3
CN_DEG = 6
N_EDGES = N_VN * VN_DEG
BATCH = 1024
NUM_ITER = 5
NUM_BITS_PER_SYMBOL = 2
CODERATE = 1.0 - N_CN / N_VN

_PHI_LO = 8.5e-8
_PHI_HI = 16.635532

NW = 32
VPW = 320
CHUNK = 16
NCHUNK = VPW // CHUNK
N_PAD = NW * VPW
M_PAD = 5120
E_PAD = CN_DEG * M_PAD
M_BLK = 256

LANES = 16
BATCH_H = BATCH // 2
NQ = BATCH_H // LANES


def _phi(x):
    x = jnp.clip(x, _PHI_LO, _PHI_HI)
    return -jnp.log(jnp.tanh(x * 0.5))


def _cn_body(z_ref, w_ref, y_ref):
    m = z_ref[...] * w_ref[...]
    mag = _phi(jnp.abs(m))
    sgn = jnp.where(m < 0.0, -1.0, 1.0)
    mag_sum = jnp.sum(mag, axis=0, keepdims=True)
    sign_tot = sgn[0:1]
    for j in range(1, CN_DEG):
        sign_tot = sign_tot * sgn[j:j + 1]
    y_ref[...] = (sign_tot * sgn) * _phi(mag_sum - mag)


def _cn_update(z3, w3):
    grid = M_PAD // M_BLK
    return pl.pallas_call(
        _cn_body,
        grid=(grid,),
        in_specs=[
            pl.BlockSpec((CN_DEG, M_BLK, BATCH_H), lambda i: (0, i, 0)),
            pl.BlockSpec((CN_DEG, M_BLK, 1), lambda i: (0, i, 0)),
        ],
        out_specs=pl.BlockSpec((CN_DEG, M_BLK, BATCH_H), lambda i: (0, i, 0)),
        out_shape=jax.ShapeDtypeStruct((CN_DEG, M_PAD, BATCH_H), jnp.float32),
    )(z3, w3)


def _vn_body(y_hbm, l_hbm, idx_hbm, z_hbm, t_hbm,
             gin0, gin1, gout0, gout1, lt0, lt1, idx_v,
             sg0, sg1, sl0, sl1, ss0, ss1):
    wid = lax.axis_index("s") * 2 + lax.axis_index("c")
    pltpu.sync_copy(idx_hbm.at[wid], idx_v)
    gin = (gin0, gin1)
    gout = (gout0, gout1)
    lt = (lt0, lt1)
    sg = (sg0, sg1)
    sl = (sl0, sl1)
    ss = (ss0, ss1)
    base0 = wid * VPW

    def issue(k, b):
        pltpu.async_copy(y_hbm.at[idx_v.at[k]], gin[b], sg[b])
        pltpu.async_copy(l_hbm.at[pl.ds(base0 + k * CHUNK, CHUNK)], lt[b], sl[b])

    for b in range(2):
        issue(b, b)

    def super_body(s, carry):
        for b in range(2):
            k = 2 * s + b
            pltpu.make_async_copy(y_hbm.at[idx_v.at[k]], gin[b], sg[b]).wait()
            pltpu.make_async_copy(
                l_hbm.at[pl.ds(base0 + k * CHUNK, CHUNK)], lt[b], sl[b]).wait()

            @pl.when(s > 0)
            def _():
                pltpu.make_async_copy(gout[b], z_hbm.at[idx_v.at[k]], ss[b]).wait()

            def q_body(q, c2):
                col = pl.ds(q * LANES, LANES)
                for vl in range(CHUNK):
                    m0 = gin[b][3 * vl + 0, col]
                    m1 = gin[b][3 * vl + 1, col]
                    m2 = gin[b][3 * vl + 2, col]
                    t = lt[b][vl, col] + (m0 + m1 + m2)
                    lt[b][vl, col] = t
                    gout[b][3 * vl + 0, col] = t - m0
                    gout[b][3 * vl + 1, col] = t - m1
                    gout[b][3 * vl + 2, col] = t - m2
                return c2

            lax.fori_loop(0, NQ, q_body, 0)
            pltpu.sync_copy(lt[b], t_hbm.at[pl.ds(base0 + k * CHUNK, CHUNK)])
            pltpu.async_copy(gout[b], z_hbm.at[idx_v.at[k]], ss[b])

            @pl.when(k + 2 < NCHUNK)
            def _():
                issue(k + 2, b)
        return carry

    lax.fori_loop(0, NCHUNK // 2, super_body, 0)
    for b in range(2):
        k = NCHUNK - 2 + b
        pltpu.make_async_copy(gout[b], z_hbm.at[idx_v.at[k]], ss[b]).wait()


@functools.lru_cache(maxsize=None)
def _make_vn_update():
    return functools.partial(
        pl.kernel,
        out_type=(
            jax.ShapeDtypeStruct((E_PAD, BATCH_H), jnp.float32),
            jax.ShapeDtypeStruct((N_PAD, BATCH_H), jnp.float32),
        ),
        mesh=plsc.VectorSubcoreMesh(core_axis_name="c", subcore_axis_name="s"),
        scratch_types=[
            pltpu.VMEM((VN_DEG * CHUNK, BATCH_H), jnp.float32),
            pltpu.VMEM((VN_DEG * CHUNK, BATCH_H), jnp.float32),
            pltpu.VMEM((VN_DEG * CHUNK, BATCH_H), jnp.float32),
            pltpu.VMEM((VN_DEG * CHUNK, BATCH_H), jnp.float32),
            pltpu.VMEM((CHUNK, BATCH_H), jnp.float32),
            pltpu.VMEM((CHUNK, BATCH_H), jnp.float32),
            pltpu.VMEM((NCHUNK, VN_DEG * CHUNK), jnp.int32),
            pltpu.SemaphoreType.DMA,
            pltpu.SemaphoreType.DMA,
            pltpu.SemaphoreType.DMA,
            pltpu.SemaphoreType.DMA,
            pltpu.SemaphoreType.DMA,
            pltpu.SemaphoreType.DMA,
        ],
    )(_vn_body)


def kernel(w_re, w_im, edge_weights, ebno_db, edge_vn, edge_cn):
    no = 1.0 / (10.0 ** (ebno_db / 10.0) * NUM_BITS_PER_SYMBOL * CODERATE)

    order = jnp.argsort(edge_cn.astype(jnp.int32), stable=True)
    rank = jnp.argsort(order, stable=True).astype(jnp.int32)
    slot = (rank % CN_DEG) * M_PAD + rank // CN_DEG
    n_pad_e = VN_DEG * (N_PAD - N_VN)
    tpe = jnp.arange(n_pad_e, dtype=jnp.int32)
    pad_slot = (tpe // (M_PAD - N_CN)) * M_PAD + N_CN + tpe % (M_PAD - N_CN)
    gidx = jnp.concatenate([slot, pad_slot]).reshape(NW, NCHUNK, VN_DEG * CHUNK)

    w_srt = jnp.take(edge_weights, order).reshape(N_CN, CN_DEG).T
    w3 = jnp.pad(w_srt, ((0, 0), (0, M_PAD - N_CN))).reshape(CN_DEG, M_PAD, 1)

    vn_update = _make_vn_update()
    zeros_y = jnp.zeros((E_PAD, BATCH_H), jnp.float32)
    l_halves, z_halves, t_halves = [], [], []
    for h in range(2):
        llr_h = (4.0 * (1.0 + w_re[h * BATCH_H:(h + 1) * BATCH_H]) / no).T
        l_h = jnp.pad(llr_h, ((0, N_PAD - N_VN), (0, 0)))
        l_halves.append(l_h)
        z0, _ = vn_update(zeros_y, l_h, gidx)
        z_halves.append(z0)
        t_halves.append(l_h)

    loss = jnp.float32(0.0)
    for _ in range(NUM_ITER):
        for h in range(2):
            y = _cn_update(z_halves[h].reshape(CN_DEG, M_PAD, BATCH_H), w3)
            z_halves[h], t_halves[h] = vn_update(
                y.reshape(E_PAD, BATCH_H), l_halves[h], gidx)
        loss = loss + 0.5 * (jnp.mean(jax.nn.softplus(-t_halves[0][:N_VN]))
                             + jnp.mean(jax.nn.softplus(-t_halves[1][:N_VN])))
    loss = loss / NUM_ITER

    c = jnp.zeros((BATCH, N_VN), dtype=jnp.float32)
    c_hat = jnp.concatenate([-t_halves[0][:N_VN].T, -t_halves[1][:N_VN].T], axis=0)
    return (c, c_hat, loss)


# speedup vs baseline: 1.3958x; 1.0273x over previous
import functools

import jax
import jax.numpy as jnp
from jax import lax
from jax.experimental import pallas as pl
from jax.experimental.pallas import tpu as pltpu
from jax.experimental.pallas import tpu_sc as plsc

N_VN = 10000
N_CN = 5000
VN_DEG = 3
CN_DEG = 6
N_EDGES = N_VN * VN_DEG
BATCH = 1024
NUM_ITER = 5
NUM_BITS_PER_SYMBOL = 2
CODERATE = 1.0 - N_CN / N_VN

_PHI_LO = 8.5e-8
_PHI_HI = 16.635532

NW = 32
VPW = 320
CHUNK = 8
NCHUNK = VPW // CHUNK
N_PAD = NW * VPW
M_PAD = 5120
E_PAD = CN_DEG * M_PAD
M_BLK = 256

LANES = 16
BATCH_H = BATCH // 2
NQ = BATCH_H // LANES


def _phi(x):
    x = jnp.clip(x, _PHI_LO, _PHI_HI)
    return -jnp.log(jnp.tanh(x * 0.5))


def _cn_body(z_ref, w_ref, y_ref):
    m = z_ref[...] * w_ref[...]
    mag = _phi(jnp.abs(m))
    sgn = jnp.where(m < 0.0, -1.0, 1.0)
    mag_sum = jnp.sum(mag, axis=0, keepdims=True)
    sign_tot = sgn[0:1]
    for j in range(1, CN_DEG):
        sign_tot = sign_tot * sgn[j:j + 1]
    y_ref[...] = (sign_tot * sgn) * _phi(mag_sum - mag)


def _cn_update(z3, w3):
    grid = M_PAD // M_BLK
    return pl.pallas_call(
        _cn_body,
        grid=(grid,),
        in_specs=[
            pl.BlockSpec((CN_DEG, M_BLK, BATCH_H), lambda i: (0, i, 0)),
            pl.BlockSpec((CN_DEG, M_BLK, 1), lambda i: (0, i, 0)),
        ],
        out_specs=pl.BlockSpec((CN_DEG, M_BLK, BATCH_H), lambda i: (0, i, 0)),
        out_shape=jax.ShapeDtypeStruct((CN_DEG, M_PAD, BATCH_H), jnp.float32),
    )(z3, w3)


def _vn_body(y_hbm, l_hbm, idx_hbm, z_hbm, t_hbm,
             gin0, gin1, gout0, gout1, li0, li1, lo0, lo1, idx_v,
             sg0, sg1, sl0, sl1, ss0, ss1, st0, st1):
    wid = lax.axis_index("s") * 2 + lax.axis_index("c")
    pltpu.sync_copy(idx_hbm.at[wid], idx_v)
    gin = (gin0, gin1)
    gout = (gout0, gout1)
    lin = (li0, li1)
    lout = (lo0, lo1)
    sg = (sg0, sg1)
    sl = (sl0, sl1)
    ss = (ss0, ss1)
    st = (st0, st1)
    base0 = wid * VPW

    def issue(k, b):
        pltpu.async_copy(y_hbm.at[idx_v.at[k]], gin[b], sg[b])
        pltpu.async_copy(l_hbm.at[pl.ds(base0 + k * CHUNK, CHUNK)], lin[b], sl[b])

    for b in range(2):
        issue(b, b)

    def super_body(s, carry):
        for b in range(2):
            k = 2 * s + b
            pltpu.make_async_copy(y_hbm.at[idx_v.at[k]], gin[b], sg[b]).wait()
            pltpu.make_async_copy(
                l_hbm.at[pl.ds(base0 + k * CHUNK, CHUNK)], lin[b], sl[b]).wait()

            @pl.when(s > 0)
            def _():
                pltpu.make_async_copy(gout[b], z_hbm.at[idx_v.at[k]], ss[b]).wait()
                pltpu.make_async_copy(
                    lout[b], t_hbm.at[pl.ds(base0 + k * CHUNK, CHUNK)],
                    st[b]).wait()

            def q_body(q, c2):
                col = pl.ds(q * LANES, LANES)
                for vl in range(CHUNK):
                    m0 = gin[b][3 * vl + 0, col]
                    m1 = gin[b][3 * vl + 1, col]
                    m2 = gin[b][3 * vl + 2, col]
                    t = lin[b][vl, col] + (m0 + m1 + m2)
                    lout[b][vl, col] = t
                    gout[b][3 * vl + 0, col] = t - m0
                    gout[b][3 * vl + 1, col] = t - m1
                    gout[b][3 * vl + 2, col] = t - m2
                return c2

            lax.fori_loop(0, NQ, q_body, 0)
            pltpu.async_copy(gout[b], z_hbm.at[idx_v.at[k]], ss[b])
            pltpu.async_copy(
                lout[b], t_hbm.at[pl.ds(base0 + k * CHUNK, CHUNK)], st[b])

            @pl.when(k + 2 < NCHUNK)
            def _():
                issue(k + 2, b)
        return carry

    lax.fori_loop(0, NCHUNK // 2, super_body, 0)
    for b in range(2):
        k = NCHUNK - 2 + b
        pltpu.make_async_copy(gout[b], z_hbm.at[idx_v.at[k]], ss[b]).wait()
        pltpu.make_async_copy(
            lout[b], t_hbm.at[pl.ds(base0 + k * CHUNK, CHUNK)], st[b]).wait()


@functools.lru_cache(maxsize=None)
def _make_vn_update():
    return functools.partial(
        pl.kernel,
        out_type=(
            jax.ShapeDtypeStruct((E_PAD, BATCH_H), jnp.float32),
            jax.ShapeDtypeStruct((N_PAD, BATCH_H), jnp.float32),
        ),
        mesh=plsc.VectorSubcoreMesh(core_axis_name="c", subcore_axis_name="s"),
        scratch_types=[
            pltpu.VMEM((VN_DEG * CHUNK, BATCH_H), jnp.float32),
            pltpu.VMEM((VN_DEG * CHUNK, BATCH_H), jnp.float32),
            pltpu.VMEM((VN_DEG * CHUNK, BATCH_H), jnp.float32),
            pltpu.VMEM((VN_DEG * CHUNK, BATCH_H), jnp.float32),
            pltpu.VMEM((CHUNK, BATCH_H), jnp.float32),
            pltpu.VMEM((CHUNK, BATCH_H), jnp.float32),
            pltpu.VMEM((CHUNK, BATCH_H), jnp.float32),
            pltpu.VMEM((CHUNK, BATCH_H), jnp.float32),
            pltpu.VMEM((NCHUNK, VN_DEG * CHUNK), jnp.int32),
            pltpu.SemaphoreType.DMA,
            pltpu.SemaphoreType.DMA,
            pltpu.SemaphoreType.DMA,
            pltpu.SemaphoreType.DMA,
            pltpu.SemaphoreType.DMA,
            pltpu.SemaphoreType.DMA,
            pltpu.SemaphoreType.DMA,
            pltpu.SemaphoreType.DMA,
        ],
    )(_vn_body)


def kernel(w_re, w_im, edge_weights, ebno_db, edge_vn, edge_cn):
    no = 1.0 / (10.0 ** (ebno_db / 10.0) * NUM_BITS_PER_SYMBOL * CODERATE)

    order = jnp.argsort(edge_cn.astype(jnp.int32), stable=True)
    rank = jnp.argsort(order, stable=True).astype(jnp.int32)
    slot = (rank % CN_DEG) * M_PAD + rank // CN_DEG
    n_pad_e = VN_DEG * (N_PAD - N_VN)
    tpe = jnp.arange(n_pad_e, dtype=jnp.int32)
    pad_slot = (tpe // (M_PAD - N_CN)) * M_PAD + N_CN + tpe % (M_PAD - N_CN)
    gidx = jnp.concatenate([slot, pad_slot]).reshape(NW, NCHUNK, VN_DEG * CHUNK)

    w_srt = jnp.take(edge_weights, order).reshape(N_CN, CN_DEG).T
    w3 = jnp.pad(w_srt, ((0, 0), (0, M_PAD - N_CN))).reshape(CN_DEG, M_PAD, 1)

    vn_update = _make_vn_update()
    zeros_y = jnp.zeros((E_PAD, BATCH_H), jnp.float32)
    l_halves, z_halves, t_halves = [], [], []
    for h in range(2):
        llr_h = (4.0 * (1.0 + w_re[h * BATCH_H:(h + 1) * BATCH_H]) / no).T
        l_h = jnp.pad(llr_h, ((0, N_PAD - N_VN), (0, 0)))
        l_halves.append(l_h)
        z0, _ = vn_update(zeros_y, l_h, gidx)
        z_halves.append(z0)
        t_halves.append(l_h)

    loss = jnp.float32(0.0)
    for _ in range(NUM_ITER):
        for h in range(2):
            y = _cn_update(z_halves[h].reshape(CN_DEG, M_PAD, BATCH_H), w3)
            z_halves[h], t_halves[h] = vn_update(
                y.reshape(E_PAD, BATCH_H), l_halves[h], gidx)
        loss = loss + 0.5 * (jnp.mean(jax.nn.softplus(-t_halves[0][:N_VN]))
                             + jnp.mean(jax.nn.softplus(-t_halves[1][:N_VN])))
    loss = loss / NUM_ITER

    c = jnp.zeros((BATCH, N_VN), dtype=jnp.float32)
    c_hat = jnp.concatenate([-t_halves[0][:N_VN].T, -t_halves[1][:N_VN].T], axis=0)
    return (c, c_hat, loss)


# speedup vs baseline: 1.3997x; 1.0028x over previous
import functools

import jax
import jax.numpy as jnp
from jax import lax
from jax.experimental import pallas as pl
from jax.experimental.pallas import tpu as pltpu
from jax.experimental.pallas import tpu_sc as plsc

N_VN = 10000
N_CN = 5000
VN_DEG = 3
CN_DEG = 6
N_EDGES = N_VN * VN_DEG
BATCH = 1024
NUM_ITER = 5
NUM_BITS_PER_SYMBOL = 2
CODERATE = 1.0 - N_CN / N_VN

_PHI_LO = 8.5e-8
_PHI_HI = 16.635532

NW = 32
VPW = 320
CHUNK = 8
NCHUNK = VPW // CHUNK
N_PAD = NW * VPW
M_PAD = 5120
E_PAD = CN_DEG * M_PAD
M_BLK = 256

LANES = 16
BATCH_H = BATCH // 2
NQ = BATCH_H // LANES


def _phi(x):
    x = jnp.clip(x, _PHI_LO, _PHI_HI)
    return -jnp.log(jnp.tanh(x * 0.5))


T_BLK = N_PAD // (M_PAD // M_BLK)


def _cn_math(z_ref, w_ref, y_ref):
    m = z_ref[...] * w_ref[...]
    mag = _phi(jnp.abs(m))
    sgn = jnp.where(m < 0.0, -1.0, 1.0)
    mag_sum = jnp.sum(mag, axis=0, keepdims=True)
    sign_tot = sgn[0:1]
    for j in range(1, CN_DEG):
        sign_tot = sign_tot * sgn[j:j + 1]
    y_ref[...] = (sign_tot * sgn) * _phi(mag_sum - mag)


def _cn_body(z_ref, w_ref, y_ref):
    _cn_math(z_ref, w_ref, y_ref)


def _cn_loss_body(z_ref, w_ref, t_ref, y_ref, p_ref):
    _cn_math(z_ref, w_ref, y_ref)
    i = pl.program_id(0)
    row = jax.lax.broadcasted_iota(jnp.int32, (T_BLK, 1), 0) + i * T_BLK
    t_safe = jnp.where(row < N_VN, t_ref[...], 1e9)
    p_ref[...] = jnp.broadcast_to(
        jnp.sum(jax.nn.softplus(-t_safe)), (1, 8, 128))


def _cn_update(z3, w3):
    grid = M_PAD // M_BLK
    return pl.pallas_call(
        _cn_body,
        grid=(grid,),
        in_specs=[
            pl.BlockSpec((CN_DEG, M_BLK, BATCH_H), lambda i: (0, i, 0)),
            pl.BlockSpec((CN_DEG, M_BLK, 1), lambda i: (0, i, 0)),
        ],
        out_specs=pl.BlockSpec((CN_DEG, M_BLK, BATCH_H), lambda i: (0, i, 0)),
        out_shape=jax.ShapeDtypeStruct((CN_DEG, M_PAD, BATCH_H), jnp.float32),
    )(z3, w3)


def _cn_update_loss(z3, w3, t_prev):
    grid = M_PAD // M_BLK
    return pl.pallas_call(
        _cn_loss_body,
        grid=(grid,),
        in_specs=[
            pl.BlockSpec((CN_DEG, M_BLK, BATCH_H), lambda i: (0, i, 0)),
            pl.BlockSpec((CN_DEG, M_BLK, 1), lambda i: (0, i, 0)),
            pl.BlockSpec((T_BLK, BATCH_H), lambda i: (i, 0)),
        ],
        out_specs=[
            pl.BlockSpec((CN_DEG, M_BLK, BATCH_H), lambda i: (0, i, 0)),
            pl.BlockSpec((1, 8, 128), lambda i: (i, 0, 0)),
        ],
        out_shape=[
            jax.ShapeDtypeStruct((CN_DEG, M_PAD, BATCH_H), jnp.float32),
            jax.ShapeDtypeStruct((grid, 8, 128), jnp.float32),
        ],
    )(z3, w3, t_prev)


def _vn_body(y_hbm, l_hbm, idx_hbm, z_hbm, t_hbm,
             gin0, gin1, gout0, gout1, li0, li1, lo0, lo1, idx_v,
             sg0, sg1, sl0, sl1, ss0, ss1, st0, st1):
    wid = lax.axis_index("s") * 2 + lax.axis_index("c")
    pltpu.sync_copy(idx_hbm.at[wid], idx_v)
    gin = (gin0, gin1)
    gout = (gout0, gout1)
    lin = (li0, li1)
    lout = (lo0, lo1)
    sg = (sg0, sg1)
    sl = (sl0, sl1)
    ss = (ss0, ss1)
    st = (st0, st1)
    base0 = wid * VPW

    def issue(k, b):
        pltpu.async_copy(y_hbm.at[idx_v.at[k]], gin[b], sg[b])
        pltpu.async_copy(l_hbm.at[pl.ds(base0 + k * CHUNK, CHUNK)], lin[b], sl[b])

    for b in range(2):
        issue(b, b)

    def super_body(s, carry):
        for b in range(2):
            k = 2 * s + b
            pltpu.make_async_copy(y_hbm.at[idx_v.at[k]], gin[b], sg[b]).wait()
            pltpu.make_async_copy(
                l_hbm.at[pl.ds(base0 + k * CHUNK, CHUNK)], lin[b], sl[b]).wait()

            @pl.when(s > 0)
            def _():
                pltpu.make_async_copy(gout[b], z_hbm.at[idx_v.at[k]], ss[b]).wait()
                pltpu.make_async_copy(
                    lout[b], t_hbm.at[pl.ds(base0 + k * CHUNK, CHUNK)],
                    st[b]).wait()

            def q_body(q, c2):
                col = pl.ds(q * LANES, LANES)
                for vl in range(CHUNK):
                    m0 = gin[b][3 * vl + 0, col]
                    m1 = gin[b][3 * vl + 1, col]
                    m2 = gin[b][3 * vl + 2, col]
                    t = lin[b][vl, col] + (m0 + m1 + m2)
                    lout[b][vl, col] = t
                    gout[b][3 * vl + 0, col] = t - m0
                    gout[b][3 * vl + 1, col] = t - m1
                    gout[b][3 * vl + 2, col] = t - m2
                return c2

            lax.fori_loop(0, NQ, q_body, 0)
            pltpu.async_copy(gout[b], z_hbm.at[idx_v.at[k]], ss[b])
            pltpu.async_copy(
                lout[b], t_hbm.at[pl.ds(base0 + k * CHUNK, CHUNK)], st[b])

            @pl.when(k + 2 < NCHUNK)
            def _():
                issue(k + 2, b)
        return carry

    lax.fori_loop(0, NCHUNK // 2, super_body, 0)
    for b in range(2):
        k = NCHUNK - 2 + b
        pltpu.make_async_copy(gout[b], z_hbm.at[idx_v.at[k]], ss[b]).wait()
        pltpu.make_async_copy(
            lout[b], t_hbm.at[pl.ds(base0 + k * CHUNK, CHUNK)], st[b]).wait()


@functools.lru_cache(maxsize=None)
def _make_vn_update():
    return functools.partial(
        pl.kernel,
        out_type=(
            jax.ShapeDtypeStruct((E_PAD, BATCH_H), jnp.float32),
            jax.ShapeDtypeStruct((N_PAD, BATCH_H), jnp.float32),
        ),
        mesh=plsc.VectorSubcoreMesh(core_axis_name="c", subcore_axis_name="s"),
        scratch_types=[
            pltpu.VMEM((VN_DEG * CHUNK, BATCH_H), jnp.float32),
            pltpu.VMEM((VN_DEG * CHUNK, BATCH_H), jnp.float32),
            pltpu.VMEM((VN_DEG * CHUNK, BATCH_H), jnp.float32),
            pltpu.VMEM((VN_DEG * CHUNK, BATCH_H), jnp.float32),
            pltpu.VMEM((CHUNK, BATCH_H), jnp.float32),
            pltpu.VMEM((CHUNK, BATCH_H), jnp.float32),
            pltpu.VMEM((CHUNK, BATCH_H), jnp.float32),
            pltpu.VMEM((CHUNK, BATCH_H), jnp.float32),
            pltpu.VMEM((NCHUNK, VN_DEG * CHUNK), jnp.int32),
            pltpu.SemaphoreType.DMA,
            pltpu.SemaphoreType.DMA,
            pltpu.SemaphoreType.DMA,
            pltpu.SemaphoreType.DMA,
            pltpu.SemaphoreType.DMA,
            pltpu.SemaphoreType.DMA,
            pltpu.SemaphoreType.DMA,
            pltpu.SemaphoreType.DMA,
        ],
    )(_vn_body)


def kernel(w_re, w_im, edge_weights, ebno_db, edge_vn, edge_cn):
    no = 1.0 / (10.0 ** (ebno_db / 10.0) * NUM_BITS_PER_SYMBOL * CODERATE)

    order = jnp.argsort(edge_cn.astype(jnp.int32), stable=True)
    rank = jnp.argsort(order, stable=True).astype(jnp.int32)
    slot = (rank % CN_DEG) * M_PAD + rank // CN_DEG
    n_pad_e = VN_DEG * (N_PAD - N_VN)
    tpe = jnp.arange(n_pad_e, dtype=jnp.int32)
    pad_slot = (tpe // (M_PAD - N_CN)) * M_PAD + N_CN + tpe % (M_PAD - N_CN)
    gidx = jnp.concatenate([slot, pad_slot]).reshape(NW, NCHUNK, VN_DEG * CHUNK)

    w_srt = jnp.take(edge_weights, order).reshape(N_CN, CN_DEG).T
    w3 = jnp.pad(w_srt, ((0, 0), (0, M_PAD - N_CN))).reshape(CN_DEG, M_PAD, 1)

    vn_update = _make_vn_update()
    zeros_y = jnp.zeros((E_PAD, BATCH_H), jnp.float32)
    l_halves, z_halves, t_halves = [], [], []
    for h in range(2):
        llr_h = (4.0 * (1.0 + w_re[h * BATCH_H:(h + 1) * BATCH_H]) / no).T
        l_h = jnp.pad(llr_h, ((0, N_PAD - N_VN), (0, 0)))
        l_halves.append(l_h)
        z0, _ = vn_update(zeros_y, l_h, gidx)
        z_halves.append(z0)
        t_halves.append(l_h)

    loss_sum = jnp.float32(0.0)
    for i in range(NUM_ITER):
        for h in range(2):
            z3 = z_halves[h].reshape(CN_DEG, M_PAD, BATCH_H)
            if i == 0:
                y = _cn_update(z3, w3)
            else:
                y, part = _cn_update_loss(z3, w3, t_halves[h])
                loss_sum = loss_sum + jnp.sum(part[:, 0, 0])
            z_halves[h], t_halves[h] = vn_update(
                y.reshape(E_PAD, BATCH_H), l_halves[h], gidx)
    for h in range(2):
        loss_sum = loss_sum + jnp.sum(jax.nn.softplus(-t_halves[h][:N_VN]))
    loss = loss_sum / (2.0 * N_VN * BATCH_H * NUM_ITER)

    c = jnp.zeros((BATCH, N_VN), dtype=jnp.float32)
    c_hat = jnp.concatenate([-t_halves[0][:N_VN].T, -t_halves[1][:N_VN].T], axis=0)
    return (c, c_hat, loss)


# speedup vs baseline: 1.4543x; 1.0390x over previous
import functools

import jax
import jax.numpy as jnp
from jax import lax
from jax.experimental import pallas as pl
from jax.experimental.pallas import tpu as pltpu
from jax.experimental.pallas import tpu_sc as plsc

N_VN = 10000
N_CN = 5000
VN_DEG = 3
CN_DEG = 6
N_EDGES = N_VN * VN_DEG
BATCH = 1024
NUM_ITER = 5
NUM_BITS_PER_SYMBOL = 2
CODERATE = 1.0 - N_CN / N_VN

_PHI_LO = 8.5e-8
_PHI_HI = 16.635532

NW = 32
VPW = 320
CHUNK = 8
NCHUNK = VPW // CHUNK
N_PAD = NW * VPW
M_PAD = 5120
E_PAD = CN_DEG * M_PAD
M_BLK = 256

LANES = 16
BATCH_H = BATCH // 2
NQ = BATCH_H // LANES


def _phi(x):
    x = jnp.clip(x, _PHI_LO, _PHI_HI)
    return -jnp.log(jnp.tanh(x * 0.5))


T_BLK = N_PAD // (M_PAD // M_BLK)


def _cn_math(z_ref, w_ref, y_ref):
    m = z_ref[...] * w_ref[...]
    mag = _phi(jnp.abs(m))
    sgn = jnp.where(m < 0.0, -1.0, 1.0)
    mag_sum = jnp.sum(mag, axis=0, keepdims=True)
    sign_tot = sgn[0:1]
    for j in range(1, CN_DEG):
        sign_tot = sign_tot * sgn[j:j + 1]
    y_ref[...] = (sign_tot * sgn) * _phi(mag_sum - mag)


def _cn_body(z_ref, w_ref, y_ref):
    _cn_math(z_ref, w_ref, y_ref)


def _cn_loss_body(z_ref, w_ref, t_ref, y_ref, p_ref):
    _cn_math(z_ref, w_ref, y_ref)
    i = pl.program_id(0)
    row = jax.lax.broadcasted_iota(jnp.int32, (T_BLK, 1), 0) + i * T_BLK
    t_safe = jnp.where(row < N_VN, t_ref[...], 1e9)
    p_ref[...] = jnp.broadcast_to(
        jnp.sum(jax.nn.softplus(-t_safe)), (1, 8, 128))


def _cn_update(z3, w3):
    grid = M_PAD // M_BLK
    return pl.pallas_call(
        _cn_body,
        grid=(grid,),
        in_specs=[
            pl.BlockSpec((CN_DEG, M_BLK, BATCH_H), lambda i: (0, i, 0)),
            pl.BlockSpec((CN_DEG, M_BLK, 1), lambda i: (0, i, 0)),
        ],
        out_specs=pl.BlockSpec((CN_DEG, M_BLK, BATCH_H), lambda i: (0, i, 0)),
        out_shape=jax.ShapeDtypeStruct((CN_DEG, M_PAD, BATCH_H), jnp.float32),
    )(z3, w3)


def _cn_update_loss(z3, w3, t_prev):
    grid = M_PAD // M_BLK
    return pl.pallas_call(
        _cn_loss_body,
        grid=(grid,),
        in_specs=[
            pl.BlockSpec((CN_DEG, M_BLK, BATCH_H), lambda i: (0, i, 0)),
            pl.BlockSpec((CN_DEG, M_BLK, 1), lambda i: (0, i, 0)),
            pl.BlockSpec((T_BLK, BATCH_H), lambda i: (i, 0)),
        ],
        out_specs=[
            pl.BlockSpec((CN_DEG, M_BLK, BATCH_H), lambda i: (0, i, 0)),
            pl.BlockSpec((1, 8, 128), lambda i: (i, 0, 0)),
        ],
        out_shape=[
            jax.ShapeDtypeStruct((CN_DEG, M_PAD, BATCH_H), jnp.float32),
            jax.ShapeDtypeStruct((grid, 8, 128), jnp.float32),
        ],
    )(z3, w3, t_prev)


def _vn_body(y_hbm, l_hbm, idx_hbm, z_hbm, t_hbm,
             gin0, gin1, gout0, gout1, li0, li1, lo0, lo1, idx_v,
             sg0, sg1, sl0, sl1, ss0, ss1, st0, st1):
    wid = lax.axis_index("s") * 2 + lax.axis_index("c")
    pltpu.sync_copy(idx_hbm.at[wid], idx_v)
    gin = (gin0, gin1)
    gout = (gout0, gout1)
    lin = (li0, li1)
    lout = (lo0, lo1)
    sg = (sg0, sg1)
    sl = (sl0, sl1)
    ss = (ss0, ss1)
    st = (st0, st1)
    base0 = wid * VPW

    def issue(k, b):
        pltpu.async_copy(y_hbm.at[idx_v.at[k]], gin[b], sg[b])
        pltpu.async_copy(l_hbm.at[pl.ds(base0 + k * CHUNK, CHUNK)], lin[b], sl[b])

    for b in range(2):
        issue(b, b)

    def super_body(s, carry):
        for b in range(2):
            k = 2 * s + b
            pltpu.make_async_copy(y_hbm.at[idx_v.at[k]], gin[b], sg[b]).wait()
            pltpu.make_async_copy(
                l_hbm.at[pl.ds(base0 + k * CHUNK, CHUNK)], lin[b], sl[b]).wait()

            @pl.when(s > 0)
            def _():
                pltpu.make_async_copy(gout[b], z_hbm.at[idx_v.at[k]], ss[b]).wait()
                pltpu.make_async_copy(
                    lout[b], t_hbm.at[pl.ds(base0 + k * CHUNK, CHUNK)],
                    st[b]).wait()

            def q_body(q, c2):
                col = pl.ds(q * LANES, LANES)
                for vl in range(CHUNK):
                    m0 = gin[b][3 * vl + 0, col]
                    m1 = gin[b][3 * vl + 1, col]
                    m2 = gin[b][3 * vl + 2, col]
                    t = lin[b][vl, col] + (m0 + m1 + m2)
                    lout[b][vl, col] = t
                    gout[b][3 * vl + 0, col] = t - m0
                    gout[b][3 * vl + 1, col] = t - m1
                    gout[b][3 * vl + 2, col] = t - m2
                return c2

            lax.fori_loop(0, NQ, q_body, 0)
            pltpu.async_copy(gout[b], z_hbm.at[idx_v.at[k]], ss[b])
            pltpu.async_copy(
                lout[b], t_hbm.at[pl.ds(base0 + k * CHUNK, CHUNK)], st[b])

            @pl.when(k + 2 < NCHUNK)
            def _():
                issue(k + 2, b)
        return carry

    lax.fori_loop(0, NCHUNK // 2, super_body, 0)
    for b in range(2):
        k = NCHUNK - 2 + b
        pltpu.make_async_copy(gout[b], z_hbm.at[idx_v.at[k]], ss[b]).wait()
        pltpu.make_async_copy(
            lout[b], t_hbm.at[pl.ds(base0 + k * CHUNK, CHUNK)], st[b]).wait()


def _vn_final_body(y_hbm, l_hbm, idx_hbm, t_hbm,
                   gin0, gin1, li0, li1, lo0, lo1, idx_v,
                   sg0, sg1, sl0, sl1, st0, st1):
    wid = lax.axis_index("s") * 2 + lax.axis_index("c")
    pltpu.sync_copy(idx_hbm.at[wid], idx_v)
    gin = (gin0, gin1)
    lin = (li0, li1)
    lout = (lo0, lo1)
    sg = (sg0, sg1)
    sl = (sl0, sl1)
    st = (st0, st1)
    base0 = wid * VPW

    def issue(k, b):
        pltpu.async_copy(y_hbm.at[idx_v.at[k]], gin[b], sg[b])
        pltpu.async_copy(l_hbm.at[pl.ds(base0 + k * CHUNK, CHUNK)], lin[b], sl[b])

    for b in range(2):
        issue(b, b)

    def super_body(s, carry):
        for b in range(2):
            k = 2 * s + b
            pltpu.make_async_copy(y_hbm.at[idx_v.at[k]], gin[b], sg[b]).wait()
            pltpu.make_async_copy(
                l_hbm.at[pl.ds(base0 + k * CHUNK, CHUNK)], lin[b], sl[b]).wait()

            @pl.when(s > 0)
            def _():
                pltpu.make_async_copy(
                    lout[b], t_hbm.at[pl.ds(base0 + k * CHUNK, CHUNK)],
                    st[b]).wait()

            def q_body(q, c2):
                col = pl.ds(q * LANES, LANES)
                for vl in range(CHUNK):
                    m0 = gin[b][3 * vl + 0, col]
                    m1 = gin[b][3 * vl + 1, col]
                    m2 = gin[b][3 * vl + 2, col]
                    lout[b][vl, col] = lin[b][vl, col] + (m0 + m1 + m2)
                return c2

            lax.fori_loop(0, NQ, q_body, 0)
            pltpu.async_copy(
                lout[b], t_hbm.at[pl.ds(base0 + k * CHUNK, CHUNK)], st[b])

            @pl.when(k + 2 < NCHUNK)
            def _():
                issue(k + 2, b)
        return carry

    lax.fori_loop(0, NCHUNK // 2, super_body, 0)
    for b in range(2):
        k = NCHUNK - 2 + b
        pltpu.make_async_copy(
            lout[b], t_hbm.at[pl.ds(base0 + k * CHUNK, CHUNK)], st[b]).wait()


@functools.lru_cache(maxsize=None)
def _make_vn_final():
    return functools.partial(
        pl.kernel,
        out_type=jax.ShapeDtypeStruct((N_PAD, BATCH_H), jnp.float32),
        mesh=plsc.VectorSubcoreMesh(core_axis_name="c", subcore_axis_name="s"),
        scratch_types=[
            pltpu.VMEM((VN_DEG * CHUNK, BATCH_H), jnp.float32),
            pltpu.VMEM((VN_DEG * CHUNK, BATCH_H), jnp.float32),
            pltpu.VMEM((CHUNK, BATCH_H), jnp.float32),
            pltpu.VMEM((CHUNK, BATCH_H), jnp.float32),
            pltpu.VMEM((CHUNK, BATCH_H), jnp.float32),
            pltpu.VMEM((CHUNK, BATCH_H), jnp.float32),
            pltpu.VMEM((NCHUNK, VN_DEG * CHUNK), jnp.int32),
            pltpu.SemaphoreType.DMA,
            pltpu.SemaphoreType.DMA,
            pltpu.SemaphoreType.DMA,
            pltpu.SemaphoreType.DMA,
            pltpu.SemaphoreType.DMA,
            pltpu.SemaphoreType.DMA,
        ],
    )(_vn_final_body)


@functools.lru_cache(maxsize=None)
def _make_vn_update():
    return functools.partial(
        pl.kernel,
        out_type=(
            jax.ShapeDtypeStruct((E_PAD, BATCH_H), jnp.float32),
            jax.ShapeDtypeStruct((N_PAD, BATCH_H), jnp.float32),
        ),
        mesh=plsc.VectorSubcoreMesh(core_axis_name="c", subcore_axis_name="s"),
        scratch_types=[
            pltpu.VMEM((VN_DEG * CHUNK, BATCH_H), jnp.float32),
            pltpu.VMEM((VN_DEG * CHUNK, BATCH_H), jnp.float32),
            pltpu.VMEM((VN_DEG * CHUNK, BATCH_H), jnp.float32),
            pltpu.VMEM((VN_DEG * CHUNK, BATCH_H), jnp.float32),
            pltpu.VMEM((CHUNK, BATCH_H), jnp.float32),
            pltpu.VMEM((CHUNK, BATCH_H), jnp.float32),
            pltpu.VMEM((CHUNK, BATCH_H), jnp.float32),
            pltpu.VMEM((CHUNK, BATCH_H), jnp.float32),
            pltpu.VMEM((NCHUNK, VN_DEG * CHUNK), jnp.int32),
            pltpu.SemaphoreType.DMA,
            pltpu.SemaphoreType.DMA,
            pltpu.SemaphoreType.DMA,
            pltpu.SemaphoreType.DMA,
            pltpu.SemaphoreType.DMA,
            pltpu.SemaphoreType.DMA,
            pltpu.SemaphoreType.DMA,
            pltpu.SemaphoreType.DMA,
        ],
    )(_vn_body)


def kernel(w_re, w_im, edge_weights, ebno_db, edge_vn, edge_cn):
    no = 1.0 / (10.0 ** (ebno_db / 10.0) * NUM_BITS_PER_SYMBOL * CODERATE)

    order = jnp.argsort(edge_cn.astype(jnp.int32), stable=True)
    rank = jnp.argsort(order, stable=True).astype(jnp.int32)
    slot = (rank % CN_DEG) * M_PAD + rank // CN_DEG
    n_pad_e = VN_DEG * (N_PAD - N_VN)
    tpe = jnp.arange(n_pad_e, dtype=jnp.int32)
    pad_slot = (tpe // (M_PAD - N_CN)) * M_PAD + N_CN + tpe % (M_PAD - N_CN)
    gidx = jnp.concatenate([slot, pad_slot]).reshape(NW, NCHUNK, VN_DEG * CHUNK)

    w_srt = jnp.take(edge_weights, order).reshape(N_CN, CN_DEG).T
    w3 = jnp.pad(w_srt, ((0, 0), (0, M_PAD - N_CN))).reshape(CN_DEG, M_PAD, 1)

    vn_update = _make_vn_update()
    vn_final = _make_vn_final()
    zeros_y = jnp.zeros((E_PAD, BATCH_H), jnp.float32)
    l_halves, z_halves, t_halves = [], [], []
    for h in range(2):
        llr_h = (4.0 * (1.0 + w_re[h * BATCH_H:(h + 1) * BATCH_H]) / no).T
        l_h = jnp.pad(llr_h, ((0, N_PAD - N_VN), (0, 0)))
        l_halves.append(l_h)
        z0, _ = vn_update(zeros_y, l_h, gidx)
        z_halves.append(z0)
        t_halves.append(l_h)

    loss_sum = jnp.float32(0.0)
    for i in range(NUM_ITER):
        for h in range(2):
            z3 = z_halves[h].reshape(CN_DEG, M_PAD, BATCH_H)
            if i == 0:
                y = _cn_update(z3, w3)
            else:
                y, part = _cn_update_loss(z3, w3, t_halves[h])
                loss_sum = loss_sum + jnp.sum(part[:, 0, 0])
            if i == NUM_ITER - 1:
                t_halves[h] = vn_final(
                    y.reshape(E_PAD, BATCH_H), l_halves[h], gidx)
            else:
                z_halves[h], t_halves[h] = vn_update(
                    y.reshape(E_PAD, BATCH_H), l_halves[h], gidx)
    for h in range(2):
        loss_sum = loss_sum + jnp.sum(jax.nn.softplus(-t_halves[h][:N_VN]))
    loss = loss_sum / (2.0 * N_VN * BATCH_H * NUM_ITER)

    c = jnp.zeros((BATCH, N_VN), dtype=jnp.float32)
    c_hat = jnp.concatenate([-t_halves[0][:N_VN].T, -t_halves[1][:N_VN].T], axis=0)
    return (c, c_hat, loss)
